# baseline (speedup 1.0000x reference)
import functools

import jax
import jax.numpy as jnp
import numpy as np
from jax import lax
from jax.experimental import pallas as pl
from jax.experimental.pallas import tpu as pltpu
from jax.experimental.pallas import tpu_sc as plsc

NUM_LEVELS = 16
F_PER_LEVEL = 2
T = 1 << 19
BASE_RES = 16
BOUND = 2.0
PER_LEVEL_SCALE = float(np.exp2(np.log2(2048 * 2 / 16) / (16 - 1)))
N_POINTS = 1048576
HIDDEN = 128
ENC_DIM = NUM_LEVELS * F_PER_LEVEL
RES = [float(np.floor(BASE_RES * PER_LEVEL_SCALE**l)) for l in range(NUM_LEVELS)]
P2 = int(np.uint32(2654435761).view(np.int32))
P3 = int(np.uint32(805459861).view(np.int32))
MASK = T - 1

NC = 2
NS = 16
NW = NC * NS
LANES = 16

C = 512
GPC = C // LANES
PW = N_POINTS // NS
HALF_L = NUM_LEVELS // 2
NCHP = PW // C
LVL = 2 * T
STAGE = LVL // NS


def _make_sc_body(lbase):
  def _sc_body(xt, tabf, resa, out, xyz, idx2, rows, wbuf, encb, resb, shr,
               gsem0, gsem1, xsem0, xsem1):
    wid = lax.axis_index("s")
    pltpu.sync_copy(resa, resb)

    xsems = [xsem0, xsem1]
    gsems = [gsem0, gsem1]

    def xyz_copies(i, b):
        base = wid * PW + i * C
        return [
            pltpu.make_async_copy(
                xt.at[pl.ds(d * N_POINTS + base, C)],
                xyz.at[pl.ds((b * 3 + d) * C, C)],
                xsems[b],
            )
            for d in range(3)
        ]

    def gather_copies(b):
        return [
            pltpu.make_async_copy(
                shr.at[idx2.at[pl.ds((b * 8 + c) * 2 * C, 2 * C)]],
                rows.at[pl.ds((b * 8 + c) * 2 * C, 2 * C)],
                gsems[b],
            )
            for c in range(8)
        ]

    def phase_body(l, carry):
        la = l + lbase
        sid = wid
        plsc.subcore_barrier()
        pltpu.sync_copy(
            tabf.at[pl.ds(la * LVL + sid * STAGE, STAGE)],
            shr.at[pl.ds(sid * STAGE, STAGE)],
        )
        plsc.subcore_barrier()

        resv = plsc.load_gather(resb, [jnp.full((LANES,), la, jnp.int32)])

        def index_fire(i, b):
            for cp in xyz_copies(i, b):
                cp.wait()

            def group(g, c2):
                g16 = g * LANES
                px = (xyz[pl.ds((b * 3 + 0) * C + g16, LANES)] + BOUND) * jnp.float32(0.25)
                py = (xyz[pl.ds((b * 3 + 1) * C + g16, LANES)] + BOUND) * jnp.float32(0.25)
                pz = (xyz[pl.ds((b * 3 + 2) * C + g16, LANES)] + BOUND) * jnp.float32(0.25)
                fposx, fposy, fposz = px * resv, py * resv, pz * resv
                ix = fposx.astype(jnp.int32)
                iy = fposy.astype(jnp.int32)
                iz = fposz.astype(jnp.int32)
                wbuf[pl.ds((b * 3 + 0) * C + g16, LANES)] = fposx - ix.astype(jnp.float32)
                wbuf[pl.ds((b * 3 + 1) * C + g16, LANES)] = fposy - iy.astype(jnp.float32)
                wbuf[pl.ds((b * 3 + 2) * C + g16, LANES)] = fposz - iz.astype(jnp.float32)
                hx = [ix, ix + 1]
                hy = [iy * P2, (iy + 1) * P2]
                hz = [iz * P3, (iz + 1) * P3]
                for c in range(8):
                    h = (hx[c & 1] ^ hy[(c >> 1) & 1] ^ hz[(c >> 2) & 1]) & MASK
                    e0 = ((h & ~127) << 1) + (h & 127)
                    rbase = (b * 8 + c) * 2 * C
                    idx2[pl.ds(rbase + g16, LANES)] = e0
                    idx2[pl.ds(rbase + C + g16, LANES)] = e0 + 128
                return c2

            lax.fori_loop(0, GPC, group, 0)
            for cp in gather_copies(b):
                cp.start()

        def acc_store(i, b):
            for cp in gather_copies(b):
                cp.wait()

            def group(g, c2):
                g16 = g * LANES
                fx = wbuf[pl.ds((b * 3 + 0) * C + g16, LANES)]
                fy = wbuf[pl.ds((b * 3 + 1) * C + g16, LANES)]
                fz = wbuf[pl.ds((b * 3 + 2) * C + g16, LANES)]
                wx = [1.0 - fx, fx]
                wy = [1.0 - fy, fy]
                wz = [1.0 - fz, fz]
                acc0 = jnp.zeros((LANES,), jnp.float32)
                acc1 = jnp.zeros((LANES,), jnp.float32)
                for c in range(8):
                    wt = wx[c & 1] * wy[(c >> 1) & 1] * wz[(c >> 2) & 1]
                    rbase = (b * 8 + c) * 2 * C
                    f0 = rows[pl.ds(rbase + g16, LANES)]
                    f1 = rows[pl.ds(rbase + C + g16, LANES)]
                    acc0 = acc0 + wt * f0
                    acc1 = acc1 + wt * f1
                encb[pl.ds((2 * b + 0) * C + g16, LANES)] = acc0
                encb[pl.ds((2 * b + 1) * C + g16, LANES)] = acc1
                return c2

            lax.fori_loop(0, GPC, group, 0)
            base = wid * PW + i * C
            pltpu.sync_copy(
                encb.at[pl.ds((2 * b + 0) * C, C)],
                out.at[pl.ds((2 * l + 0) * N_POINTS + base, C)],
            )
            pltpu.sync_copy(
                encb.at[pl.ds((2 * b + 1) * C, C)],
                out.at[pl.ds((2 * l + 1) * N_POINTS + base, C)],
            )

        for cp in xyz_copies(0, 0):
            cp.start()

        def pair_body(j, c2):
            i0 = 2 * j
            i1 = i0 + 1
            index_fire(i0, 0)

            for cp in xyz_copies(i1, 1):
                cp.start()

            @pl.when(j > 0)
            def _():
                acc_store(i0 - 1, 1)

            index_fire(i1, 1)

            @pl.when(i1 + 1 < NCHP)
            def _():
                for cp in xyz_copies(i1 + 1, 0):
                    cp.start()

            acc_store(i0, 0)
            return c2

        lax.fori_loop(0, NCHP // 2, pair_body, 0)
        acc_store(NCHP - 1, 1)
        return carry

    lax.fori_loop(0, HALF_L, phase_body, 0)

  return _sc_body


def _sc_encode(xt, tabf, lbase):
    mesh = plsc.VectorSubcoreMesh(core_axis_name="c", subcore_axis_name="s",
                                  num_cores=1)
    f = functools.partial(
        pl.kernel,
        mesh=mesh,
        compiler_params=pltpu.CompilerParams(needs_layout_passes=False),
        out_type=jax.ShapeDtypeStruct((HALF_L * 2 * N_POINTS,), jnp.float32),
        scratch_types=[
            pltpu.VMEM((2 * 3 * C,), jnp.float32),
            pltpu.VMEM((2 * 8 * 2 * C,), jnp.int32),
            pltpu.VMEM((2 * 8 * 2 * C,), jnp.float32),
            pltpu.VMEM((2 * 3 * C,), jnp.float32),
            pltpu.VMEM((2 * 2 * C,), jnp.float32),
            pltpu.VMEM((LANES,), jnp.float32),
            pltpu.VMEM_SHARED((LVL,), jnp.float32),
            pltpu.SemaphoreType.DMA,
            pltpu.SemaphoreType.DMA,
            pltpu.SemaphoreType.DMA,
            pltpu.SemaphoreType.DMA,
        ],
    )(_make_sc_body(lbase))
    return f(xt, tabf, jnp.asarray(RES, dtype=jnp.float32))


def _mlp_body(enca_ref, encb_ref, w0a_ref, w0b_ref, w1_ref, o_ref):
    h = jax.lax.dot_general(
        enca_ref[...], w0a_ref[...], (((0,), (0,)), ((), ())),
        preferred_element_type=jnp.float32,
    ) + jax.lax.dot_general(
        encb_ref[...], w0b_ref[...], (((0,), (0,)), ((), ())),
        preferred_element_type=jnp.float32,
    )
    h = jnp.maximum(h, 0.0)
    o = jnp.dot(h, w1_ref[...], preferred_element_type=jnp.float32)
    o_ref[...] = jax.nn.sigmoid(o) * 2.0 - 1.0


def _mlp(encA, encB, W0, W1):
    NB = 8192
    grid = (N_POINTS // NB,)
    HL2 = HALF_L * 2
    return pl.pallas_call(
        _mlp_body,
        grid=grid,
        in_specs=[
            pl.BlockSpec((HL2, NB), lambda i: (0, i)),
            pl.BlockSpec((HL2, NB), lambda i: (0, i)),
            pl.BlockSpec((HL2, HIDDEN), lambda i: (0, 0)),
            pl.BlockSpec((HL2, HIDDEN), lambda i: (0, 0)),
            pl.BlockSpec((HIDDEN, 3), lambda i: (0, 0)),
        ],
        out_specs=pl.BlockSpec((NB, 3), lambda i: (i, 0)),
        out_shape=jax.ShapeDtypeStruct((N_POINTS, 3), jnp.float32),
    )(encA, encB, W0[:HL2], W0[HL2:], W1)


def kernel(input, table, W0, W1):
    xt = jnp.transpose(input).reshape(-1)
    tabf = (
        table.reshape(NUM_LEVELS, T // 128, 128, F_PER_LEVEL)
        .transpose(0, 1, 3, 2)
        .reshape(NUM_LEVELS * T * F_PER_LEVEL)
    )
    encA = _sc_encode(xt, tabf, 0).reshape(HALF_L * 2, N_POINTS)
    encB = _sc_encode(xt, tabf, HALF_L).reshape(HALF_L * 2, N_POINTS)
    return _mlp(encA, encB, W0, W1)


# speedup vs baseline: 1.2204x; 1.2204x over previous
import functools

import jax
import jax.numpy as jnp
import numpy as np
from jax import lax
from jax.experimental import pallas as pl
from jax.experimental.pallas import tpu as pltpu
from jax.experimental.pallas import tpu_sc as plsc

NUM_LEVELS = 16
F_PER_LEVEL = 2
T = 1 << 19
BASE_RES = 16
BOUND = 2.0
PER_LEVEL_SCALE = float(np.exp2(np.log2(2048 * 2 / 16) / (16 - 1)))
N_POINTS = 1048576
HIDDEN = 128
ENC_DIM = NUM_LEVELS * F_PER_LEVEL
RES = [float(np.floor(BASE_RES * PER_LEVEL_SCALE**l)) for l in range(NUM_LEVELS)]
P2 = int(np.uint32(2654435761).view(np.int32))
P3 = int(np.uint32(805459861).view(np.int32))
MASK = T - 1

NC = 2
NS = 16
NW = NC * NS
LANES = 16

C = 512
GPC = C // LANES
PW = N_POINTS // NW
NCHP = PW // C
LVL = 2 * T
STAGE = LVL // NS


def _sc_body(xt, tabf, resa, out, xyz, idx2, rows, wbuf, encb, resb, shr,
             gsem0, gsem1, xsem0, xsem1):
    cid = lax.axis_index("c")
    sid = lax.axis_index("s")
    wid = sid * NC + cid
    pltpu.sync_copy(resa, resb)

    xsems = [xsem0, xsem1]
    gsems = [gsem0, gsem1]

    def xyz_copies(i, b):
        base = wid * PW + i * C
        return [
            pltpu.make_async_copy(
                xt.at[pl.ds(d * N_POINTS + base, C)],
                xyz.at[pl.ds((b * 3 + d) * C, C)],
                xsems[b],
            )
            for d in range(3)
        ]

    def gather_copies(b):
        return [
            pltpu.make_async_copy(
                shr.at[idx2.at[pl.ds((b * 8 + c) * 2 * C, 2 * C)]],
                rows.at[pl.ds((b * 8 + c) * 2 * C, 2 * C)],
                gsems[b],
            )
            for c in range(8)
        ]

    def phase_body(l, carry):
        plsc.subcore_barrier()
        pltpu.sync_copy(
            tabf.at[pl.ds(l * LVL + sid * STAGE, STAGE)],
            shr.at[pl.ds(sid * STAGE, STAGE)],
        )
        plsc.subcore_barrier()

        resv = plsc.load_gather(resb, [jnp.full((LANES,), l, jnp.int32)])

        def index_fire(i, b):
            for cp in xyz_copies(i, b):
                cp.wait()

            @plsc.parallel_loop(0, GPC, unroll=2)
            def group(g):
                g16 = g * LANES
                px = (xyz[pl.ds((b * 3 + 0) * C + g16, LANES)] + BOUND) * jnp.float32(0.25)
                py = (xyz[pl.ds((b * 3 + 1) * C + g16, LANES)] + BOUND) * jnp.float32(0.25)
                pz = (xyz[pl.ds((b * 3 + 2) * C + g16, LANES)] + BOUND) * jnp.float32(0.25)
                fposx, fposy, fposz = px * resv, py * resv, pz * resv
                ix = fposx.astype(jnp.int32)
                iy = fposy.astype(jnp.int32)
                iz = fposz.astype(jnp.int32)
                wbuf[pl.ds((b * 3 + 0) * C + g16, LANES)] = fposx - ix.astype(jnp.float32)
                wbuf[pl.ds((b * 3 + 1) * C + g16, LANES)] = fposy - iy.astype(jnp.float32)
                wbuf[pl.ds((b * 3 + 2) * C + g16, LANES)] = fposz - iz.astype(jnp.float32)
                hy0 = iy * P2
                hz0 = iz * P3
                hx = [ix, ix + 1]
                hy = [hy0, hy0 + P2]
                hz = [hz0, hz0 + P3]
                for c in range(8):
                    h = (hx[c & 1] ^ hy[(c >> 1) & 1] ^ hz[(c >> 2) & 1]) & MASK
                    e0 = h + (h & ~127)
                    rbase = (b * 8 + c) * 2 * C
                    idx2[pl.ds(rbase + g16, LANES)] = e0
                    idx2[pl.ds(rbase + C + g16, LANES)] = e0 + 128

            for cp in gather_copies(b):
                cp.start()

        def acc_store(i, b):
            for cp in gather_copies(b):
                cp.wait()

            @plsc.parallel_loop(0, GPC, unroll=2)
            def group(g):
                g16 = g * LANES
                fx = wbuf[pl.ds((b * 3 + 0) * C + g16, LANES)]
                fy = wbuf[pl.ds((b * 3 + 1) * C + g16, LANES)]
                fz = wbuf[pl.ds((b * 3 + 2) * C + g16, LANES)]
                wx = [1.0 - fx, fx]
                wy = [1.0 - fy, fy]
                wz = [1.0 - fz, fz]
                wyz = [wy[0] * wz[0], wy[1] * wz[0], wy[0] * wz[1], wy[1] * wz[1]]
                acc0 = jnp.zeros((LANES,), jnp.float32)
                acc1 = jnp.zeros((LANES,), jnp.float32)
                for c in range(8):
                    wt = wx[c & 1] * wyz[c >> 1]
                    rbase = (b * 8 + c) * 2 * C
                    f0 = rows[pl.ds(rbase + g16, LANES)]
                    f1 = rows[pl.ds(rbase + C + g16, LANES)]
                    acc0 = acc0 + wt * f0
                    acc1 = acc1 + wt * f1
                encb[pl.ds((2 * b + 0) * C + g16, LANES)] = acc0
                encb[pl.ds((2 * b + 1) * C + g16, LANES)] = acc1

            base = wid * PW + i * C
            pltpu.sync_copy(
                encb.at[pl.ds((2 * b + 0) * C, C)],
                out.at[pl.ds((2 * l + 0) * N_POINTS + base, C)],
            )
            pltpu.sync_copy(
                encb.at[pl.ds((2 * b + 1) * C, C)],
                out.at[pl.ds((2 * l + 1) * N_POINTS + base, C)],
            )

        for cp in xyz_copies(0, 0):
            cp.start()

        def pair_body(j, c2):
            i0 = 2 * j
            i1 = i0 + 1
            index_fire(i0, 0)

            for cp in xyz_copies(i1, 1):
                cp.start()

            @pl.when(j > 0)
            def _():
                acc_store(i0 - 1, 1)

            index_fire(i1, 1)

            @pl.when(i1 + 1 < NCHP)
            def _():
                for cp in xyz_copies(i1 + 1, 0):
                    cp.start()

            acc_store(i0, 0)
            return c2

        lax.fori_loop(0, NCHP // 2, pair_body, 0)
        acc_store(NCHP - 1, 1)
        return carry

    lax.fori_loop(0, NUM_LEVELS, phase_body, 0)


def _sc_encode(xt, tabf):
    mesh = plsc.VectorSubcoreMesh(core_axis_name="c", subcore_axis_name="s")
    f = functools.partial(
        pl.kernel,
        mesh=mesh,
        compiler_params=pltpu.CompilerParams(needs_layout_passes=False),
        out_type=jax.ShapeDtypeStruct((ENC_DIM * N_POINTS,), jnp.float32),
        scratch_types=[
            pltpu.VMEM((2 * 3 * C,), jnp.float32),
            pltpu.VMEM((2 * 8 * 2 * C,), jnp.int32),
            pltpu.VMEM((2 * 8 * 2 * C,), jnp.float32),
            pltpu.VMEM((2 * 3 * C,), jnp.float32),
            pltpu.VMEM((2 * 2 * C,), jnp.float32),
            pltpu.VMEM((LANES,), jnp.float32),
            pltpu.VMEM_SHARED((LVL,), jnp.float32),
            pltpu.SemaphoreType.DMA,
            pltpu.SemaphoreType.DMA,
            pltpu.SemaphoreType.DMA,
            pltpu.SemaphoreType.DMA,
        ],
    )(_sc_body)
    return f(xt, tabf, jnp.asarray(RES, dtype=jnp.float32))


def _mlp_body(enc_ref, w0_ref, w1_ref, o_ref):
    h = jax.lax.dot_general(
        enc_ref[...], w0_ref[...], (((0,), (0,)), ((), ())),
        preferred_element_type=jnp.float32,
    )
    h = jnp.maximum(h, 0.0)
    o = jnp.dot(h, w1_ref[...], preferred_element_type=jnp.float32)
    o_ref[...] = jax.nn.sigmoid(o) * 2.0 - 1.0


def _mlp(enc32, W0, W1):
    NB = 8192
    grid = (N_POINTS // NB,)
    return pl.pallas_call(
        _mlp_body,
        grid=grid,
        in_specs=[
            pl.BlockSpec((ENC_DIM, NB), lambda i: (0, i)),
            pl.BlockSpec((ENC_DIM, HIDDEN), lambda i: (0, 0)),
            pl.BlockSpec((HIDDEN, 3), lambda i: (0, 0)),
        ],
        out_specs=pl.BlockSpec((NB, 3), lambda i: (i, 0)),
        out_shape=jax.ShapeDtypeStruct((N_POINTS, 3), jnp.float32),
    )(enc32, W0, W1)


def kernel(input, table, W0, W1):
    xt = jnp.transpose(input).reshape(-1)
    tabf = (
        table.reshape(NUM_LEVELS, T // 128, 128, F_PER_LEVEL)
        .transpose(0, 1, 3, 2)
        .reshape(NUM_LEVELS * T * F_PER_LEVEL)
    )
    enc = _sc_encode(xt, tabf)
    enc32 = enc.reshape(ENC_DIM, N_POINTS)
    return _mlp(enc32, W0, W1)


# speedup vs baseline: 1.2217x; 1.0011x over previous
import functools

import jax
import jax.numpy as jnp
import numpy as np
from jax import lax
from jax.experimental import pallas as pl
from jax.experimental.pallas import tpu as pltpu
from jax.experimental.pallas import tpu_sc as plsc

NUM_LEVELS = 16
F_PER_LEVEL = 2
T = 1 << 19
BASE_RES = 16
BOUND = 2.0
PER_LEVEL_SCALE = float(np.exp2(np.log2(2048 * 2 / 16) / (16 - 1)))
N_POINTS = 1048576
HIDDEN = 128
ENC_DIM = NUM_LEVELS * F_PER_LEVEL
RES = [float(np.floor(BASE_RES * PER_LEVEL_SCALE**l)) for l in range(NUM_LEVELS)]
P2 = int(np.uint32(2654435761).view(np.int32))
P3 = int(np.uint32(805459861).view(np.int32))
MASK = T - 1

NC = 2
NS = 16
NW = NC * NS
LANES = 16

C = 512
GPC = C // LANES
PW = N_POINTS // NW
NCHP = PW // C
LVL = 2 * T
STAGE = LVL // NS


def _sc_body(xt, tabf, resa, out, xyz, idx2, rows, wbuf, encb, resb, shr,
             gsem0, gsem1, xsem0, xsem1, ssem0, ssem1):
    cid = lax.axis_index("c")
    sid = lax.axis_index("s")
    wid = sid * NC + cid
    pltpu.sync_copy(resa, resb)

    xsems = [xsem0, xsem1]
    gsems = [gsem0, gsem1]
    ssems = [ssem0, ssem1]

    def xyz_copies(i, b):
        base = wid * PW + i * C
        return [
            pltpu.make_async_copy(
                xt.at[pl.ds(d * N_POINTS + base, C)],
                xyz.at[pl.ds((b * 3 + d) * C, C)],
                xsems[b],
            )
            for d in range(3)
        ]

    def gather_copies(b):
        return [
            pltpu.make_async_copy(
                shr.at[idx2.at[pl.ds((b * 8 + c) * 2 * C, 2 * C)]],
                rows.at[pl.ds((b * 8 + c) * 2 * C, 2 * C)],
                gsems[b],
            )
            for c in range(8)
        ]

    def phase_body(l, carry):
        plsc.subcore_barrier()
        pltpu.sync_copy(
            tabf.at[pl.ds(l * LVL + sid * STAGE, STAGE)],
            shr.at[pl.ds(sid * STAGE, STAGE)],
        )
        plsc.subcore_barrier()

        resv = plsc.load_gather(resb, [jnp.full((LANES,), l, jnp.int32)])

        def index_fire(i, b):
            for cp in xyz_copies(i, b):
                cp.wait()

            @plsc.parallel_loop(0, GPC, unroll=2)
            def group(g):
                g16 = g * LANES
                px = (xyz[pl.ds((b * 3 + 0) * C + g16, LANES)] + BOUND) * jnp.float32(0.25)
                py = (xyz[pl.ds((b * 3 + 1) * C + g16, LANES)] + BOUND) * jnp.float32(0.25)
                pz = (xyz[pl.ds((b * 3 + 2) * C + g16, LANES)] + BOUND) * jnp.float32(0.25)
                fposx, fposy, fposz = px * resv, py * resv, pz * resv
                ix = fposx.astype(jnp.int32)
                iy = fposy.astype(jnp.int32)
                iz = fposz.astype(jnp.int32)
                wbuf[pl.ds((b * 3 + 0) * C + g16, LANES)] = fposx - ix.astype(jnp.float32)
                wbuf[pl.ds((b * 3 + 1) * C + g16, LANES)] = fposy - iy.astype(jnp.float32)
                wbuf[pl.ds((b * 3 + 2) * C + g16, LANES)] = fposz - iz.astype(jnp.float32)
                hy0 = iy * P2
                hz0 = iz * P3
                hx = [ix, ix + 1]
                hy = [hy0, hy0 + P2]
                hz = [hz0, hz0 + P3]
                for c in range(8):
                    h = (hx[c & 1] ^ hy[(c >> 1) & 1] ^ hz[(c >> 2) & 1]) & MASK
                    e0 = h + (h & ~127)
                    rbase = (b * 8 + c) * 2 * C
                    idx2[pl.ds(rbase + g16, LANES)] = e0
                    idx2[pl.ds(rbase + C + g16, LANES)] = e0 + 128

            for cp in gather_copies(b):
                cp.start()

        def store_copies(i, b):
            base = wid * PW + i * C
            return [
                pltpu.make_async_copy(
                    encb.at[pl.ds((2 * b + f) * C, C)],
                    out.at[pl.ds((2 * l + f) * N_POINTS + base, C)],
                    ssems[b],
                )
                for f in range(2)
            ]

        def acc_store(i, b):
            for cp in gather_copies(b):
                cp.wait()

            @pl.when(i >= 2)
            def _():
                for cp in store_copies(i, b):
                    cp.wait()

            @plsc.parallel_loop(0, GPC, unroll=2)
            def group(g):
                g16 = g * LANES
                fx = wbuf[pl.ds((b * 3 + 0) * C + g16, LANES)]
                fy = wbuf[pl.ds((b * 3 + 1) * C + g16, LANES)]
                fz = wbuf[pl.ds((b * 3 + 2) * C + g16, LANES)]
                wx = [1.0 - fx, fx]
                wy = [1.0 - fy, fy]
                wz = [1.0 - fz, fz]
                wyz = [wy[0] * wz[0], wy[1] * wz[0], wy[0] * wz[1], wy[1] * wz[1]]
                acc0 = jnp.zeros((LANES,), jnp.float32)
                acc1 = jnp.zeros((LANES,), jnp.float32)
                for c in range(8):
                    wt = wx[c & 1] * wyz[c >> 1]
                    rbase = (b * 8 + c) * 2 * C
                    f0 = rows[pl.ds(rbase + g16, LANES)]
                    f1 = rows[pl.ds(rbase + C + g16, LANES)]
                    acc0 = acc0 + wt * f0
                    acc1 = acc1 + wt * f1
                encb[pl.ds((2 * b + 0) * C + g16, LANES)] = acc0
                encb[pl.ds((2 * b + 1) * C + g16, LANES)] = acc1

            for cp in store_copies(i, b):
                cp.start()

        for cp in xyz_copies(0, 0):
            cp.start()

        def pair_body(j, c2):
            i0 = 2 * j
            i1 = i0 + 1
            index_fire(i0, 0)

            for cp in xyz_copies(i1, 1):
                cp.start()

            @pl.when(j > 0)
            def _():
                acc_store(i0 - 1, 1)

            index_fire(i1, 1)

            @pl.when(i1 + 1 < NCHP)
            def _():
                for cp in xyz_copies(i1 + 1, 0):
                    cp.start()

            acc_store(i0, 0)
            return c2

        lax.fori_loop(0, NCHP // 2, pair_body, 0)
        acc_store(NCHP - 1, 1)
        for cp in store_copies(NCHP - 2, 0):
            cp.wait()
        for cp in store_copies(NCHP - 1, 1):
            cp.wait()
        return carry

    lax.fori_loop(0, NUM_LEVELS, phase_body, 0)


def _sc_encode(xt, tabf):
    mesh = plsc.VectorSubcoreMesh(core_axis_name="c", subcore_axis_name="s")
    f = functools.partial(
        pl.kernel,
        mesh=mesh,
        compiler_params=pltpu.CompilerParams(needs_layout_passes=False),
        out_type=jax.ShapeDtypeStruct((ENC_DIM * N_POINTS,), jnp.float32),
        scratch_types=[
            pltpu.VMEM((2 * 3 * C,), jnp.float32),
            pltpu.VMEM((2 * 8 * 2 * C,), jnp.int32),
            pltpu.VMEM((2 * 8 * 2 * C,), jnp.float32),
            pltpu.VMEM((2 * 3 * C,), jnp.float32),
            pltpu.VMEM((2 * 2 * C,), jnp.float32),
            pltpu.VMEM((LANES,), jnp.float32),
            pltpu.VMEM_SHARED((LVL,), jnp.float32),
            pltpu.SemaphoreType.DMA,
            pltpu.SemaphoreType.DMA,
            pltpu.SemaphoreType.DMA,
            pltpu.SemaphoreType.DMA,
            pltpu.SemaphoreType.DMA,
            pltpu.SemaphoreType.DMA,
        ],
    )(_sc_body)
    return f(xt, tabf, jnp.asarray(RES, dtype=jnp.float32))


def _mlp_body(enc_ref, w0_ref, w1_ref, o_ref):
    h = jax.lax.dot_general(
        enc_ref[...], w0_ref[...], (((0,), (0,)), ((), ())),
        preferred_element_type=jnp.float32,
    )
    h = jnp.maximum(h, 0.0)
    o = jnp.dot(h, w1_ref[...], preferred_element_type=jnp.float32)
    o_ref[...] = jax.nn.sigmoid(o) * 2.0 - 1.0


def _mlp(enc32, W0, W1):
    NB = 8192
    grid = (N_POINTS // NB,)
    return pl.pallas_call(
        _mlp_body,
        grid=grid,
        in_specs=[
            pl.BlockSpec((ENC_DIM, NB), lambda i: (0, i)),
            pl.BlockSpec((ENC_DIM, HIDDEN), lambda i: (0, 0)),
            pl.BlockSpec((HIDDEN, 3), lambda i: (0, 0)),
        ],
        out_specs=pl.BlockSpec((NB, 3), lambda i: (i, 0)),
        out_shape=jax.ShapeDtypeStruct((N_POINTS, 3), jnp.float32),
    )(enc32, W0, W1)


def kernel(input, table, W0, W1):
    xt = jnp.transpose(input).reshape(-1)
    tabf = (
        table.reshape(NUM_LEVELS, T // 128, 128, F_PER_LEVEL)
        .transpose(0, 1, 3, 2)
        .reshape(NUM_LEVELS * T * F_PER_LEVEL)
    )
    enc = _sc_encode(xt, tabf)
    enc32 = enc.reshape(ENC_DIM, N_POINTS)
    return _mlp(enc32, W0, W1)
